# baseline (speedup 1.0000x reference)
import jax
import jax.numpy as jnp
from jax import lax
from jax.experimental import pallas as pl
from jax.experimental.pallas import tpu as pltpu
import jax.experimental.pallas.tpu_sc as plsc

N = 10000
E = 320000
F = 128
G = 64

NC = 2
NS = 16
NW = NC * NS
EW = E // NW
K = 96
NCHUNK = EW // K
TAIL = EW - NCHUNK * K
RA = 624
RB = 640

NBUF = 3
LEAD = 1

PACK = 16384


def _aggr_body(x_hbm, packed_hbm, zeros_hbm, out0_hbm, out1_hbm,
               packed_all, srcb, dstb, rows_v, tsrc, tdst, trows, aggr_sh,
               *sems):
    gsem = sems[:NBUF]
    ssem = sems[NBUF:]
    c = lax.axis_index("c")
    s = lax.axis_index("s")
    wid = s * NC + c

    isem = sems[2 * NBUF]

    @pl.when(s < NS - 1)
    def _():
        pltpu.async_copy(zeros_hbm.at[pl.ds(0, RA)],
                         aggr_sh.at[pl.ds(s * RA, RA)], isem)

    @pl.when(s == NS - 1)
    def _():
        pltpu.async_copy(zeros_hbm, aggr_sh.at[pl.ds((NS - 1) * RA, RB)],
                         isem)

    pltpu.sync_copy(packed_hbm.at[pl.ds(wid * EW, EW)], packed_all)

    def refill(i, b):
        for j in range(K // 16):
            v = packed_all[pl.ds(i * K + j * 16, 16)]
            srcb[b, pl.ds(j * 16, 16)] = lax.shift_right_logical(v, 14)
            dstb[b, pl.ds(j * 16, 16)] = jnp.bitwise_and(v, PACK - 1)
        pltpu.async_copy(x_hbm.at[srcb.at[b]], rows_v.at[b], gsem[b])

    def gather_wait(b):
        pltpu.make_async_copy(x_hbm.at[pl.ds(0, K)], rows_v.at[b],
                              gsem[b]).wait()

    def scatter_wait(b):
        pltpu.make_async_copy(rows_v.at[b], aggr_sh.at[pl.ds(0, K)],
                              ssem[b]).wait()

    def step(i, b, do_wait_s):
        gather_wait(b)
        pltpu.async_copy(rows_v.at[b], aggr_sh.at[dstb.at[b]], ssem[b],
                         add=True)
        nb = (b + LEAD) % NBUF
        if do_wait_s:
            scatter_wait(nb)

        @pl.when(i + LEAD < NCHUNK)
        def _():
            refill(i + LEAD, nb)

    for i in range(LEAD):
        refill(i, i)

    @pl.when(s < NS - 1)
    def _():
        pltpu.make_async_copy(zeros_hbm.at[pl.ds(0, RA)],
                              aggr_sh.at[pl.ds(0, RA)], isem).wait()

    @pl.when(s == NS - 1)
    def _():
        pltpu.make_async_copy(zeros_hbm, aggr_sh.at[pl.ds(0, RB)],
                              isem).wait()

    plsc.subcore_barrier()

    for i in range(NBUF):
        step(i, i % NBUF, i + LEAD - NBUF >= 0)

    def group(j, carry):
        for b in range(NBUF):
            step(j * NBUF + b, b, True)
        return carry

    lax.fori_loop(1, NCHUNK // NBUF, group, 0, unroll=False)
    for i in range((NCHUNK // NBUF) * NBUF, NCHUNK):
        step(i, i % NBUF, True)
    for i in range(NCHUNK - (NBUF - LEAD), NCHUNK):
        scatter_wait(i % NBUF)

    v = packed_all[pl.ds(NCHUNK * K, TAIL)]
    tsrc[...] = lax.shift_right_logical(v, 14)
    tdst[...] = jnp.bitwise_and(v, PACK - 1)
    pltpu.async_copy(x_hbm.at[tsrc], trows, gsem[0])
    pltpu.make_async_copy(x_hbm.at[pl.ds(0, TAIL)], trows, gsem[0]).wait()
    pltpu.sync_copy(trows, aggr_sh.at[tdst], add=True)

    plsc.subcore_barrier()
    @pl.when((s < NS - 1) & (c == 0))
    def _():
        pltpu.sync_copy(aggr_sh.at[pl.ds(s * RA, RA)],
                        out0_hbm.at[pl.ds(s * RA, RA)])

    @pl.when((s < NS - 1) & (c == 1))
    def _():
        pltpu.sync_copy(aggr_sh.at[pl.ds(s * RA, RA)],
                        out1_hbm.at[pl.ds(s * RA, RA)])

    @pl.when((s == NS - 1) & (c == 0))
    def _():
        pltpu.sync_copy(aggr_sh.at[pl.ds((NS - 1) * RA, RB)],
                        out0_hbm.at[pl.ds((NS - 1) * RA, RB)])

    @pl.when((s == NS - 1) & (c == 1))
    def _():
        pltpu.sync_copy(aggr_sh.at[pl.ds((NS - 1) * RA, RB)],
                        out1_hbm.at[pl.ds((NS - 1) * RA, RB)])


@jax.jit
def _sc_aggregate(x, packed, zeros_rows):
    mesh = plsc.VectorSubcoreMesh(core_axis_name="c", subcore_axis_name="s")
    f = pl.kernel(
        _aggr_body,
        out_type=[jax.ShapeDtypeStruct((N, F), jnp.float32),
                  jax.ShapeDtypeStruct((N, F), jnp.float32)],
        mesh=mesh,
        scratch_types=[
            pltpu.VMEM((EW,), jnp.int32),
            pltpu.VMEM((NBUF, K), jnp.int32),
            pltpu.VMEM((NBUF, K), jnp.int32),
            pltpu.VMEM((NBUF, K, F), jnp.float32),
            pltpu.VMEM((TAIL,), jnp.int32),
            pltpu.VMEM((TAIL,), jnp.int32),
            pltpu.VMEM((TAIL, F), jnp.float32),
            pltpu.VMEM_SHARED((N, F), jnp.float32),
        ] + [pltpu.SemaphoreType.DMA] * (2 * NBUF + 1),
    )
    return f(x, packed, zeros_rows)


def _layer_tc_body(x_ref, a0_ref, a1_ref, w1_ref, b1_ref, w2_ref, b2_ref,
                   g_ref, be_ref, o_ref):
    h = x_ref[...] + a0_ref[...] + a1_ref[...]
    h = jnp.dot(h, w1_ref[...], preferred_element_type=jnp.float32) + b1_ref[...]
    h = jnp.maximum(h, 0.0)
    y = jnp.dot(h, w2_ref[...], preferred_element_type=jnp.float32) + b2_ref[...]
    y = jnp.maximum(y, 0.0)
    mean = jnp.mean(y, axis=0, keepdims=True)
    d = y - mean
    var = jnp.mean(d * d, axis=0, keepdims=True)
    o_ref[...] = d * lax.rsqrt(var + 1e-5) * g_ref[...] + be_ref[...]


@jax.jit
def _layer_tc(x, a0, a1, w1, b1, w2, b2, gamma, beta):
    return pl.pallas_call(
        _layer_tc_body,
        out_shape=jax.ShapeDtypeStruct((N, F), jnp.float32),
    )(x, a0, a1, w1, b1.reshape(1, F), w2, b2.reshape(1, F),
      gamma.reshape(1, F), beta.reshape(1, F))


def _layer3_tc_body(x_ref, a0_ref, a1_ref, w1_ref, b1_ref, w2_ref, b2_ref,
                    g_ref, be_ref, batch_ref, o_ref):
    h = x_ref[...] + a0_ref[...] + a1_ref[...]
    h = jnp.dot(h, w1_ref[...], preferred_element_type=jnp.float32) + b1_ref[...]
    h = jnp.maximum(h, 0.0)
    y = jnp.dot(h, w2_ref[...], preferred_element_type=jnp.float32) + b2_ref[...]
    y = jnp.maximum(y, 0.0)
    mean = jnp.mean(y, axis=0, keepdims=True)
    d = y - mean
    var = jnp.mean(d * d, axis=0, keepdims=True)
    hn = d * lax.rsqrt(var + 1e-5) * g_ref[...] + be_ref[...]
    gid = lax.broadcasted_iota(jnp.int32, (G, N), 0)
    onehot = (gid == batch_ref[...]).astype(jnp.float32)
    o_ref[...] = jnp.dot(onehot, hn, preferred_element_type=jnp.float32)


@jax.jit
def _layer3_tc(x, a0, a1, w1, b1, w2, b2, gamma, beta, batch2):
    return pl.pallas_call(
        _layer3_tc_body,
        out_shape=jax.ShapeDtypeStruct((G, F), jnp.float32),
    )(x, a0, a1, w1, b1.reshape(1, F), w2, b2.reshape(1, F),
      gamma.reshape(1, F), beta.reshape(1, F), batch2)


def kernel(x, edge_index, batch, W1_1, b1_1, W1_2, b1_2, gamma1, beta1,
           W2_1, b2_1, W2_2, b2_2, gamma2, beta2,
           W3_1, b3_1, W3_2, b3_2, gamma3, beta3):
    packed = edge_index[0] * PACK + edge_index[1]
    zeros_rows = jnp.zeros((RB, F), dtype=jnp.float32)
    batch2 = batch.reshape(1, N)

    a0, a1 = _sc_aggregate(x, packed, zeros_rows)
    h = _layer_tc(x, a0, a1, W1_1, b1_1, W1_2, b1_2, gamma1, beta1)
    a0, a1 = _sc_aggregate(h, packed, zeros_rows)
    h = _layer_tc(h, a0, a1, W2_1, b2_1, W2_2, b2_2, gamma2, beta2)
    a0, a1 = _sc_aggregate(h, packed, zeros_rows)
    return _layer3_tc(h, a0, a1, W3_1, b3_1, W3_2, b3_2, gamma3, beta3,
                      batch2)


# speedup vs baseline: 1.3896x; 1.3896x over previous
import jax
import jax.numpy as jnp
from jax import lax
from jax.experimental import pallas as pl
from jax.experimental.pallas import tpu as pltpu
import jax.experimental.pallas.tpu_sc as plsc

N = 10000
E = 320000
F = 128
G = 64

NC = 2
NS = 16
NW = NC * NS
EW = E // NW
K = 96
NCHUNK = EW // K
TAIL = EW - NCHUNK * K
RA = 624
RB = 640

NBUF = 3
LEAD = 2

PACK = 16384


def _aggr_body(x_hbm, packed_hbm, zeros_hbm, out0_hbm, out1_hbm,
               packed_all, srcb, dstb, rows_v, tsrc, tdst, trows, aggr_sh,
               *sems):
    gsem = sems[:NBUF]
    ssem = sems[NBUF:]
    c = lax.axis_index("c")
    s = lax.axis_index("s")
    wid = s * NC + c

    isem = sems[2 * NBUF]

    @pl.when(s < NS - 1)
    def _():
        pltpu.async_copy(zeros_hbm.at[pl.ds(0, RA)],
                         aggr_sh.at[pl.ds(s * RA, RA)], isem)

    @pl.when(s == NS - 1)
    def _():
        pltpu.async_copy(zeros_hbm, aggr_sh.at[pl.ds((NS - 1) * RA, RB)],
                         isem)

    pltpu.sync_copy(packed_hbm.at[pl.ds(wid * EW, EW)], packed_all)

    def refill(i, b):
        for j in range(K // 16):
            v = packed_all[pl.ds(i * K + j * 16, 16)]
            srcb[b, pl.ds(j * 16, 16)] = lax.shift_right_logical(v, 14)
            dstb[b, pl.ds(j * 16, 16)] = jnp.bitwise_and(v, PACK - 1)
        pltpu.async_copy(x_hbm.at[srcb.at[b, pl.ds(0, K // 2)]],
                         rows_v.at[b, pl.ds(0, K // 2)], gsem[b])
        pltpu.async_copy(x_hbm.at[srcb.at[b, pl.ds(K // 2, K // 2)]],
                         rows_v.at[b, pl.ds(K // 2, K // 2)], gsem[b])

    def gather_wait(b):
        pltpu.make_async_copy(x_hbm.at[pl.ds(0, K)], rows_v.at[b],
                              gsem[b]).wait()

    def scatter_wait(b):
        pltpu.make_async_copy(rows_v.at[b], aggr_sh.at[pl.ds(0, K)],
                              ssem[b]).wait()

    def step(i, b, do_wait_s):
        gather_wait(b)
        pltpu.async_copy(rows_v.at[b], aggr_sh.at[dstb.at[b]], ssem[b],
                         add=True)
        nb = (b + LEAD) % NBUF
        if do_wait_s:
            scatter_wait(nb)

        @pl.when(i + LEAD < NCHUNK)
        def _():
            refill(i + LEAD, nb)

    for i in range(LEAD):
        refill(i, i)

    @pl.when(s < NS - 1)
    def _():
        pltpu.make_async_copy(zeros_hbm.at[pl.ds(0, RA)],
                              aggr_sh.at[pl.ds(0, RA)], isem).wait()

    @pl.when(s == NS - 1)
    def _():
        pltpu.make_async_copy(zeros_hbm, aggr_sh.at[pl.ds(0, RB)],
                              isem).wait()

    plsc.subcore_barrier()

    for i in range(NBUF):
        step(i, i % NBUF, i + LEAD - NBUF >= 0)

    def group(j, carry):
        for b in range(NBUF):
            step(j * NBUF + b, b, True)
        return carry

    lax.fori_loop(1, NCHUNK // NBUF, group, 0, unroll=False)
    for i in range((NCHUNK // NBUF) * NBUF, NCHUNK):
        step(i, i % NBUF, True)
    for i in range(NCHUNK - (NBUF - LEAD), NCHUNK):
        scatter_wait(i % NBUF)

    v = packed_all[pl.ds(NCHUNK * K, TAIL)]
    tsrc[...] = lax.shift_right_logical(v, 14)
    tdst[...] = jnp.bitwise_and(v, PACK - 1)
    pltpu.async_copy(x_hbm.at[tsrc], trows, gsem[0])
    pltpu.make_async_copy(x_hbm.at[pl.ds(0, TAIL)], trows, gsem[0]).wait()
    pltpu.sync_copy(trows, aggr_sh.at[tdst], add=True)

    plsc.subcore_barrier()
    @pl.when((s < NS - 1) & (c == 0))
    def _():
        pltpu.sync_copy(aggr_sh.at[pl.ds(s * RA, RA)],
                        out0_hbm.at[pl.ds(s * RA, RA)])

    @pl.when((s < NS - 1) & (c == 1))
    def _():
        pltpu.sync_copy(aggr_sh.at[pl.ds(s * RA, RA)],
                        out1_hbm.at[pl.ds(s * RA, RA)])

    @pl.when((s == NS - 1) & (c == 0))
    def _():
        pltpu.sync_copy(aggr_sh.at[pl.ds((NS - 1) * RA, RB)],
                        out0_hbm.at[pl.ds((NS - 1) * RA, RB)])

    @pl.when((s == NS - 1) & (c == 1))
    def _():
        pltpu.sync_copy(aggr_sh.at[pl.ds((NS - 1) * RA, RB)],
                        out1_hbm.at[pl.ds((NS - 1) * RA, RB)])


@jax.jit
def _sc_aggregate(x, packed, zeros_rows):
    mesh = plsc.VectorSubcoreMesh(core_axis_name="c", subcore_axis_name="s")
    f = pl.kernel(
        _aggr_body,
        out_type=[jax.ShapeDtypeStruct((N, F), jnp.float32),
                  jax.ShapeDtypeStruct((N, F), jnp.float32)],
        mesh=mesh,
        scratch_types=[
            pltpu.VMEM((EW,), jnp.int32),
            pltpu.VMEM((NBUF, K), jnp.int32),
            pltpu.VMEM((NBUF, K), jnp.int32),
            pltpu.VMEM((NBUF, K, F), jnp.float32),
            pltpu.VMEM((TAIL,), jnp.int32),
            pltpu.VMEM((TAIL,), jnp.int32),
            pltpu.VMEM((TAIL, F), jnp.float32),
            pltpu.VMEM_SHARED((N, F), jnp.float32),
        ] + [pltpu.SemaphoreType.DMA] * (2 * NBUF + 1),
    )
    return f(x, packed, zeros_rows)


def _layer_tc_body(x_ref, a0_ref, a1_ref, w1_ref, b1_ref, w2_ref, b2_ref,
                   g_ref, be_ref, o_ref):
    h = x_ref[...] + a0_ref[...] + a1_ref[...]
    h = jnp.dot(h, w1_ref[...], preferred_element_type=jnp.float32) + b1_ref[...]
    h = jnp.maximum(h, 0.0)
    y = jnp.dot(h, w2_ref[...], preferred_element_type=jnp.float32) + b2_ref[...]
    y = jnp.maximum(y, 0.0)
    mean = jnp.mean(y, axis=0, keepdims=True)
    d = y - mean
    var = jnp.mean(d * d, axis=0, keepdims=True)
    o_ref[...] = d * lax.rsqrt(var + 1e-5) * g_ref[...] + be_ref[...]


@jax.jit
def _layer_tc(x, a0, a1, w1, b1, w2, b2, gamma, beta):
    return pl.pallas_call(
        _layer_tc_body,
        out_shape=jax.ShapeDtypeStruct((N, F), jnp.float32),
    )(x, a0, a1, w1, b1.reshape(1, F), w2, b2.reshape(1, F),
      gamma.reshape(1, F), beta.reshape(1, F))


def _layer3_tc_body(x_ref, a0_ref, a1_ref, w1_ref, b1_ref, w2_ref, b2_ref,
                    g_ref, be_ref, batch_ref, o_ref):
    h = x_ref[...] + a0_ref[...] + a1_ref[...]
    h = jnp.dot(h, w1_ref[...], preferred_element_type=jnp.float32) + b1_ref[...]
    h = jnp.maximum(h, 0.0)
    y = jnp.dot(h, w2_ref[...], preferred_element_type=jnp.float32) + b2_ref[...]
    y = jnp.maximum(y, 0.0)
    mean = jnp.mean(y, axis=0, keepdims=True)
    d = y - mean
    var = jnp.mean(d * d, axis=0, keepdims=True)
    hn = d * lax.rsqrt(var + 1e-5) * g_ref[...] + be_ref[...]
    gid = lax.broadcasted_iota(jnp.int32, (G, N), 0)
    onehot = (gid == batch_ref[...]).astype(jnp.float32)
    o_ref[...] = jnp.dot(onehot, hn, preferred_element_type=jnp.float32)


@jax.jit
def _layer3_tc(x, a0, a1, w1, b1, w2, b2, gamma, beta, batch2):
    return pl.pallas_call(
        _layer3_tc_body,
        out_shape=jax.ShapeDtypeStruct((G, F), jnp.float32),
    )(x, a0, a1, w1, b1.reshape(1, F), w2, b2.reshape(1, F),
      gamma.reshape(1, F), beta.reshape(1, F), batch2)


def kernel(x, edge_index, batch, W1_1, b1_1, W1_2, b1_2, gamma1, beta1,
           W2_1, b2_1, W2_2, b2_2, gamma2, beta2,
           W3_1, b3_1, W3_2, b3_2, gamma3, beta3):
    packed = edge_index[0] * PACK + edge_index[1]
    zeros_rows = jnp.zeros((RB, F), dtype=jnp.float32)
    batch2 = batch.reshape(1, N)

    a0, a1 = _sc_aggregate(x, packed, zeros_rows)
    h = _layer_tc(x, a0, a1, W1_1, b1_1, W1_2, b1_2, gamma1, beta1)
    a0, a1 = _sc_aggregate(h, packed, zeros_rows)
    h = _layer_tc(h, a0, a1, W2_1, b2_1, W2_2, b2_2, gamma2, beta2)
    a0, a1 = _sc_aggregate(h, packed, zeros_rows)
    return _layer3_tc(h, a0, a1, W3_1, b3_1, W3_2, b3_2, gamma3, beta3,
                      batch2)
